# baseline (speedup 1.0000x reference)
import functools

import jax
import jax.numpy as jnp
from jax import lax
from jax.experimental import pallas as pl
from jax.experimental.pallas import tpu as pltpu
from jax.experimental.pallas import tpu_sc as plsc

N = 10000
E = 320000
D = 128

_NC = 2
_NS = 16
_NW = _NC * _NS
_EPW = E // _NW
_K = 80
_SCH = 25
_NSUP = _EPW // (_SCH * _K)
_L = 16


def _mm_body(x_ref, w_ref, o_ref):
    o_ref[...] = jnp.dot(x_ref[...], w_ref[...],
                         preferred_element_type=jnp.float32)


def _add_body(a_ref, b_ref, o_ref):
    o_ref[...] = a_ref[...] + b_ref[...]


def _sc_scatter(support, src, dst, wts):
    mesh = plsc.VectorSubcoreMesh(core_axis_name="c", subcore_axis_name="s")

    @functools.partial(
        pl.kernel,
        mesh=mesh,
        out_type=jax.ShapeDtypeStruct((_NC, N, D), jnp.float32),
        scratch_types=[
            pltpu.VMEM((_SCH, _K), jnp.int32),
            pltpu.VMEM((_SCH, _K), jnp.int32),
            pltpu.VMEM((_SCH * _K,), jnp.float32),
            pltpu.VMEM((3, _K, D), jnp.float32),
            pltpu.VMEM_SHARED((N, D), jnp.float32),
            pltpu.SemaphoreType.DMA((3,)),
            pltpu.SemaphoreType.DMA((3,)),
        ],
    )
    def k(support_hbm, src_hbm, dst_hbm, w_hbm, out_hbm,
          src_v, dst_v, w_v, rows_v, acc, gsem, ssem):
        c = lax.axis_index("c")
        s = lax.axis_index("s")
        wid = c * _NS + s

        def zero_body(i, carry):
            for f in range(D // _L):
                rows_v[0, i, pl.ds(f * _L, _L)] = jnp.zeros((_L,), jnp.float32)
            return carry

        lax.fori_loop(0, _K, zero_body, 0)

        @pl.when(s < 10)
        def _zero():
            for r in range(12):
                pltpu.sync_copy(rows_v.at[0],
                                acc.at[pl.ds(s * 1000 + r * _K, _K)])
            pltpu.sync_copy(rows_v.at[0].at[pl.ds(0, 40)],
                            acc.at[pl.ds(s * 1000 + 960, 40)])

        plsc.subcore_barrier()

        def super_body(u, carry):
            pltpu.sync_copy(src_hbm.at[wid, u], src_v)
            pltpu.sync_copy(dst_hbm.at[wid, u], dst_v)
            pltpu.sync_copy(w_hbm.at[wid, u], w_v)

            pltpu.async_copy(support_hbm.at[src_v.at[0]],
                             rows_v.at[0], gsem.at[0])

            def chunk_body(j, carry2):
                b = j % 3
                nb = (j + 1) % 3

                @pl.when(j >= 2)
                def _wait_scatter():
                    pltpu.make_async_copy(
                        rows_v.at[nb], acc.at[dst_v.at[j - 2]],
                        ssem.at[nb]).wait()

                @pl.when(j < _SCH - 1)
                def _prefetch():
                    pltpu.async_copy(support_hbm.at[src_v.at[j + 1]],
                                     rows_v.at[nb], gsem.at[nb])

                pltpu.make_async_copy(support_hbm.at[src_v.at[j]],
                                      rows_v.at[b], gsem.at[b]).wait()

                def group_body(g, inner):
                    wvec = w_v[pl.ds(j * _K + g * _L, _L)]
                    for lane in range(_L):
                        wb = jnp.full((_L,), wvec[lane])
                        row = g * _L + lane
                        for f in range(D // _L):
                            rows_v[b, row, pl.ds(f * _L, _L)] = (
                                rows_v[b, row, pl.ds(f * _L, _L)] * wb)
                    return inner

                lax.fori_loop(0, _K // _L, group_body, 0)
                pltpu.async_copy(rows_v.at[b], acc.at[dst_v.at[j]],
                                 ssem.at[b], add=True)
                return carry2

            lax.fori_loop(0, _SCH, chunk_body, 0)

            for t in (_SCH - 2, _SCH - 1):
                pltpu.make_async_copy(rows_v.at[t % 3],
                                      acc.at[dst_v.at[t]],
                                      ssem.at[t % 3]).wait()
            return carry

        lax.fori_loop(0, _NSUP, super_body, 0)

        plsc.subcore_barrier()

        @pl.when(s < 10)
        def _drain():
            pltpu.sync_copy(acc.at[pl.ds(s * 1000, 1000)],
                            out_hbm.at[c, pl.ds(s * 1000, 1000)])

    return k(support, src, dst, wts)


def kernel(edge_index, edge_weight, input_feature, W):
    blk = N // 10
    support = pl.pallas_call(
        _mm_body,
        grid=(10,),
        in_specs=[pl.BlockSpec((blk, D), lambda i: (i, 0)),
                  pl.BlockSpec((D, D), lambda i: (0, 0))],
        out_specs=pl.BlockSpec((blk, D), lambda i: (i, 0)),
        out_shape=jax.ShapeDtypeStruct((N, D), jnp.float32),
    )(input_feature, W)

    src = edge_index[0].reshape(_NW, _NSUP, _SCH, _K)
    dst = edge_index[1].reshape(_NW, _NSUP, _SCH, _K)
    w = edge_weight.reshape(_NW, _NSUP, _SCH * _K)

    partials = _sc_scatter(support, src, dst, w)

    out = pl.pallas_call(
        _add_body,
        grid=(10,),
        in_specs=[pl.BlockSpec((blk, D), lambda i: (i, 0)),
                  pl.BlockSpec((blk, D), lambda i: (i, 0))],
        out_specs=pl.BlockSpec((blk, D), lambda i: (i, 0)),
        out_shape=jax.ShapeDtypeStruct((N, D), jnp.float32),
    )(partials[0], partials[1])
    return out


# speedup vs baseline: 1.6816x; 1.6816x over previous
import functools

import jax
import jax.numpy as jnp
from jax import lax
from jax.experimental import pallas as pl
from jax.experimental.pallas import tpu as pltpu
from jax.experimental.pallas import tpu_sc as plsc

N = 10000
E = 320000
D = 128

_NC = 2
_NS = 16
_NW = _NC * _NS
_EPW = E // _NW
_K = 80
_SCH = 25
_NSUP = _EPW // (_SCH * _K)
_L = 16


def _mm_body(x_ref, w_ref, o_ref):
    o_ref[...] = jnp.dot(x_ref[...], w_ref[...],
                         preferred_element_type=jnp.float32)


def _add_body(a_ref, b_ref, o_ref):
    o_ref[...] = a_ref[...] + b_ref[...]


def _sc_scatter(support, src, dst, wts):
    mesh = plsc.VectorSubcoreMesh(core_axis_name="c", subcore_axis_name="s")

    @functools.partial(
        pl.kernel,
        mesh=mesh,
        out_type=jax.ShapeDtypeStruct((_NC, N, D), jnp.float32),
        scratch_types=[
            pltpu.VMEM((_SCH, _K), jnp.int32),
            pltpu.VMEM((_SCH, _K), jnp.int32),
            pltpu.VMEM((_SCH * _K,), jnp.float32),
            pltpu.VMEM((_K, D), jnp.float32),
            pltpu.VMEM((_K, D), jnp.float32),
            pltpu.VMEM_SHARED((N, D), jnp.float32),
            pltpu.SemaphoreType.DMA,
            pltpu.SemaphoreType.DMA,
            pltpu.SemaphoreType.DMA,
            pltpu.SemaphoreType.DMA,
        ],
    )
    def k(support_hbm, src_hbm, dst_hbm, w_hbm, out_hbm,
          src_v, dst_v, w_v, rows0, rows1, acc, gsem0, gsem1, ssem0, ssem1):
        c = lax.axis_index("c")
        s = lax.axis_index("s")
        wid = c * _NS + s

        def zero_body(i, carry):
            for f in range(D // _L):
                rows0[i, pl.ds(f * _L, _L)] = jnp.zeros((_L,), jnp.float32)
            return carry

        lax.fori_loop(0, _K, zero_body, 0)

        @pl.when(s < 10)
        def _zero():
            for r in range(12):
                pltpu.sync_copy(rows0, acc.at[pl.ds(s * 1000 + r * _K, _K)])
            pltpu.sync_copy(rows0.at[pl.ds(0, 40)],
                            acc.at[pl.ds(s * 1000 + 960, 40)])

        plsc.subcore_barrier()

        def scale(rows, j):
            def group_body(g, inner):
                wvec = w_v[pl.ds(j * _K + g * _L, _L)]
                for lane in range(_L):
                    wb = jnp.full((_L,), wvec[lane])
                    row = g * _L + lane
                    for f in range(D // _L):
                        rows[row, pl.ds(f * _L, _L)] = (
                            rows[row, pl.ds(f * _L, _L)] * wb)
                return inner

            lax.fori_loop(0, _K // _L, group_body, 0)

        def super_body(u, carry):
            pltpu.sync_copy(src_hbm.at[wid, u], src_v)
            pltpu.sync_copy(dst_hbm.at[wid, u], dst_v)
            pltpu.sync_copy(w_hbm.at[wid, u], w_v)

            pltpu.async_copy(support_hbm.at[src_v.at[0]], rows0, gsem0)

            def pair_body(m, carry2):
                a = 2 * m
                pltpu.make_async_copy(support_hbm.at[src_v.at[a]],
                                      rows0, gsem0).wait()
                scale(rows0, a)

                @pl.when(m >= 1)
                def _recycle1():
                    pltpu.make_async_copy(rows1, acc.at[dst_v.at[a - 1]],
                                          ssem1).wait()

                pltpu.async_copy(support_hbm.at[src_v.at[a + 1]],
                                 rows1, gsem1)
                pltpu.async_copy(rows0, acc.at[dst_v.at[a]],
                                 ssem0, add=True)

                pltpu.make_async_copy(support_hbm.at[src_v.at[a + 1]],
                                      rows1, gsem1).wait()
                scale(rows1, a + 1)
                pltpu.make_async_copy(rows0, acc.at[dst_v.at[a]],
                                      ssem0).wait()
                pltpu.async_copy(support_hbm.at[src_v.at[a + 2]],
                                 rows0, gsem0)
                pltpu.async_copy(rows1, acc.at[dst_v.at[a + 1]],
                                 ssem1, add=True)
                return carry2

            lax.fori_loop(0, (_SCH - 1) // 2, pair_body, 0)

            t = _SCH - 1
            pltpu.make_async_copy(support_hbm.at[src_v.at[t]],
                                  rows0, gsem0).wait()
            scale(rows0, t)
            pltpu.make_async_copy(rows1, acc.at[dst_v.at[t - 1]],
                                  ssem1).wait()
            pltpu.async_copy(rows0, acc.at[dst_v.at[t]], ssem0, add=True)
            pltpu.make_async_copy(rows0, acc.at[dst_v.at[t]], ssem0).wait()
            return carry

        lax.fori_loop(0, _NSUP, super_body, 0)

        plsc.subcore_barrier()

        @pl.when(s < 10)
        def _drain():
            pltpu.sync_copy(acc.at[pl.ds(s * 1000, 1000)],
                            out_hbm.at[c, pl.ds(s * 1000, 1000)])

    return k(support, src, dst, wts)


def kernel(edge_index, edge_weight, input_feature, W):
    blk = N // 10
    support = pl.pallas_call(
        _mm_body,
        grid=(10,),
        in_specs=[pl.BlockSpec((blk, D), lambda i: (i, 0)),
                  pl.BlockSpec((D, D), lambda i: (0, 0))],
        out_specs=pl.BlockSpec((blk, D), lambda i: (i, 0)),
        out_shape=jax.ShapeDtypeStruct((N, D), jnp.float32),
    )(input_feature, W)

    src = edge_index[0].reshape(_NW, _NSUP, _SCH, _K)
    dst = edge_index[1].reshape(_NW, _NSUP, _SCH, _K)
    w = edge_weight.reshape(_NW, _NSUP, _SCH * _K)

    partials = _sc_scatter(support, src, dst, w)

    out = pl.pallas_call(
        _add_body,
        grid=(10,),
        in_specs=[pl.BlockSpec((blk, D), lambda i: (i, 0)),
                  pl.BlockSpec((blk, D), lambda i: (i, 0))],
        out_specs=pl.BlockSpec((blk, D), lambda i: (i, 0)),
        out_shape=jax.ShapeDtypeStruct((N, D), jnp.float32),
    )(partials[0], partials[1])
    return out


# speedup vs baseline: 2.3531x; 1.3993x over previous
import functools

import jax
import jax.numpy as jnp
from jax import lax
from jax.experimental import pallas as pl
from jax.experimental.pallas import tpu as pltpu
from jax.experimental.pallas import tpu_sc as plsc

N = 10000
E = 320000
D = 128

_NC = 2
_NS = 16
_NW = _NC * _NS
_EPW = E // _NW
_K = 80
_SCH = 25
_NSUP = _EPW // (_SCH * _K)
_L = 16


def _mm_body(x_ref, w_ref, o_ref):
    o_ref[...] = jnp.dot(x_ref[...], w_ref[...],
                         preferred_element_type=jnp.float32)


def _add_body(a_ref, b_ref, o_ref):
    o_ref[...] = a_ref[...] + b_ref[...]


def _sc_scatter(support, src, dst, wts):
    mesh = plsc.VectorSubcoreMesh(core_axis_name="c", subcore_axis_name="s")

    @functools.partial(
        pl.kernel,
        mesh=mesh,
        out_type=jax.ShapeDtypeStruct((_NC, N, D), jnp.float32),
        scratch_types=[
            pltpu.VMEM((_SCH, _K), jnp.int32),
            pltpu.VMEM((_SCH, _K), jnp.int32),
            pltpu.VMEM((_SCH * _K,), jnp.float32),
            pltpu.VMEM((_K, D), jnp.float32),
            pltpu.VMEM((_K, D), jnp.float32),
            pltpu.VMEM((_K, D), jnp.float32),
            pltpu.VMEM_SHARED((N, D), jnp.float32),
            pltpu.SemaphoreType.DMA,
            pltpu.SemaphoreType.DMA,
            pltpu.SemaphoreType.DMA,
            pltpu.SemaphoreType.DMA,
            pltpu.SemaphoreType.DMA,
            pltpu.SemaphoreType.DMA,
        ],
    )
    def k(support_hbm, src_hbm, dst_hbm, w_hbm, out_hbm,
          src_v, dst_v, w_v, rows0, rows1, rows2, acc,
          gsem0, gsem1, gsem2, ssem0, ssem1, ssem2):
        c = lax.axis_index("c")
        s = lax.axis_index("s")
        wid = c * _NS + s

        def zero_body(i, carry):
            for f in range(D // _L):
                rows0[i, pl.ds(f * _L, _L)] = jnp.zeros((_L,), jnp.float32)
            return carry

        lax.fori_loop(0, _K, zero_body, 0)

        @pl.when(s < 10)
        def _zero():
            for r in range(12):
                pltpu.sync_copy(rows0, acc.at[pl.ds(s * 1000 + r * _K, _K)])
            pltpu.sync_copy(rows0.at[pl.ds(0, 40)],
                            acc.at[pl.ds(s * 1000 + 960, 40)])

        plsc.subcore_barrier()

        def scale(rows, j):
            def group_body(g, inner):
                wvec = w_v[pl.ds(j * _K + g * _L, _L)]
                for lane in range(_L):
                    wb = jnp.full((_L,), wvec[lane])
                    row = g * _L + lane
                    for f in range(D // _L):
                        rows[row, pl.ds(f * _L, _L)] = (
                            rows[row, pl.ds(f * _L, _L)] * wb)
                return inner

            lax.fori_loop(0, _K // _L, group_body, 0)

        def super_body(u, carry):
            pltpu.sync_copy(src_hbm.at[wid, u], src_v)
            pltpu.sync_copy(dst_hbm.at[wid, u], dst_v)
            pltpu.sync_copy(w_hbm.at[wid, u], w_v)

            pltpu.async_copy(support_hbm.at[src_v.at[0]], rows0, gsem0)

            bufs = (rows0, rows1, rows2)
            gsems = (gsem0, gsem1, gsem2)
            ssems = (ssem0, ssem1, ssem2)

            def triple_body(m, carry2):
                for k in range(3):
                    c = 3 * m + k
                    nk = (k + 1) % 3

                    if k == 2:
                        pltpu.make_async_copy(
                            bufs[nk], acc.at[dst_v.at[c - 2]],
                            ssems[nk]).wait()
                    else:
                        @pl.when(m >= 1)
                        def _recycle(c=c, nk=nk):
                            pltpu.make_async_copy(
                                bufs[nk], acc.at[dst_v.at[c - 2]],
                                ssems[nk]).wait()

                    pltpu.async_copy(support_hbm.at[src_v.at[c + 1]],
                                     bufs[nk], gsems[nk])

                    pltpu.make_async_copy(support_hbm.at[src_v.at[c]],
                                          bufs[k], gsems[k]).wait()
                    scale(bufs[k], c)
                    pltpu.async_copy(bufs[k], acc.at[dst_v.at[c]],
                                     ssems[k], add=True)
                return carry2

            lax.fori_loop(0, (_SCH - 1) // 3, triple_body, 0)

            t = _SCH - 1
            pltpu.make_async_copy(support_hbm.at[src_v.at[t]],
                                  rows0, gsem0).wait()
            scale(rows0, t)
            pltpu.async_copy(rows0, acc.at[dst_v.at[t]], ssem0, add=True)
            pltpu.make_async_copy(rows1, acc.at[dst_v.at[t - 2]],
                                  ssem1).wait()
            pltpu.make_async_copy(rows2, acc.at[dst_v.at[t - 1]],
                                  ssem2).wait()
            pltpu.make_async_copy(rows0, acc.at[dst_v.at[t]], ssem0).wait()
            return carry

        lax.fori_loop(0, _NSUP, super_body, 0)

        plsc.subcore_barrier()

        @pl.when(s < 10)
        def _drain():
            pltpu.sync_copy(acc.at[pl.ds(s * 1000, 1000)],
                            out_hbm.at[c, pl.ds(s * 1000, 1000)])

    return k(support, src, dst, wts)


def kernel(edge_index, edge_weight, input_feature, W):
    blk = N // 10
    support = pl.pallas_call(
        _mm_body,
        grid=(10,),
        in_specs=[pl.BlockSpec((blk, D), lambda i: (i, 0)),
                  pl.BlockSpec((D, D), lambda i: (0, 0))],
        out_specs=pl.BlockSpec((blk, D), lambda i: (i, 0)),
        out_shape=jax.ShapeDtypeStruct((N, D), jnp.float32),
    )(input_feature, W)

    src = edge_index[0].reshape(_NW, _NSUP, _SCH, _K)
    dst = edge_index[1].reshape(_NW, _NSUP, _SCH, _K)
    w = edge_weight.reshape(_NW, _NSUP, _SCH * _K)

    partials = _sc_scatter(support, src, dst, w)

    out = pl.pallas_call(
        _add_body,
        grid=(10,),
        in_specs=[pl.BlockSpec((blk, D), lambda i: (i, 0)),
                  pl.BlockSpec((blk, D), lambda i: (i, 0))],
        out_specs=pl.BlockSpec((blk, D), lambda i: (i, 0)),
        out_shape=jax.ShapeDtypeStruct((N, D), jnp.float32),
    )(partials[0], partials[1])
    return out


# speedup vs baseline: 2.4188x; 1.0279x over previous
import functools

import jax
import jax.numpy as jnp
from jax import lax
from jax.experimental import pallas as pl
from jax.experimental.pallas import tpu as pltpu
from jax.experimental.pallas import tpu_sc as plsc

N = 10000
E = 320000
D = 128

_NC = 2
_NS = 16
_NW = _NC * _NS
_EPW = E // _NW
_K = 80
_SCH = 25
_NSUP = _EPW // (_SCH * _K)
_L = 16


def _mm_body(x_ref, w_ref, o_ref):
    o_ref[...] = jnp.dot(x_ref[...], w_ref[...],
                         preferred_element_type=jnp.float32)


def _add_body(a_ref, b_ref, o_ref):
    o_ref[...] = a_ref[...] + b_ref[...]


def _sc_scatter(support, src, dst, wts):
    mesh = plsc.VectorSubcoreMesh(core_axis_name="c", subcore_axis_name="s")

    @functools.partial(
        pl.kernel,
        mesh=mesh,
        out_type=jax.ShapeDtypeStruct((_NC, N, D), jnp.float32),
        scratch_types=[
            pltpu.VMEM((_SCH, _K), jnp.int32),
            pltpu.VMEM((_SCH, _K), jnp.int32),
            pltpu.VMEM((_SCH * _K,), jnp.float32),
            pltpu.VMEM((_K, D), jnp.float32),
            pltpu.VMEM((_K, D), jnp.float32),
            pltpu.VMEM((_K, D), jnp.float32),
            pltpu.VMEM_SHARED((N, D), jnp.float32),
            pltpu.SemaphoreType.DMA,
            pltpu.SemaphoreType.DMA,
            pltpu.SemaphoreType.DMA,
            pltpu.SemaphoreType.DMA,
            pltpu.SemaphoreType.DMA,
            pltpu.SemaphoreType.DMA,
        ],
    )
    def k(support_hbm, src_hbm, dst_hbm, w_hbm, out_hbm,
          src_v, dst_v, w_v, rows0, rows1, rows2, acc,
          gsem0, gsem1, gsem2, ssem0, ssem1, ssem2):
        c = lax.axis_index("c")
        s = lax.axis_index("s")
        wid = c * _NS + s

        def zero_body(i, carry):
            for f in range(D // _L):
                rows0[i, pl.ds(f * _L, _L)] = jnp.zeros((_L,), jnp.float32)
            return carry

        lax.fori_loop(0, _K, zero_body, 0)

        @pl.when(s < 10)
        def _zero():
            for r in range(12):
                pltpu.sync_copy(rows0, acc.at[pl.ds(s * 1000 + r * _K, _K)])
            pltpu.sync_copy(rows0.at[pl.ds(0, 40)],
                            acc.at[pl.ds(s * 1000 + 960, 40)])

        plsc.subcore_barrier()

        def scale(rows, j):
            def group_body(g, inner):
                wvec = w_v[pl.ds(j * _K + g * _L, _L)]
                for lane in range(_L):
                    wb = jnp.full((_L,), wvec[lane])
                    row = g * _L + lane
                    for f in range(D // _L):
                        rows[row, pl.ds(f * _L, _L)] = (
                            rows[row, pl.ds(f * _L, _L)] * wb)
                return inner

            lax.fori_loop(0, _K // _L, group_body, 0)

        def super_body(u, carry):
            pltpu.async_copy(src_hbm.at[wid, u], src_v, gsem0)
            pltpu.async_copy(dst_hbm.at[wid, u], dst_v, gsem1)
            pltpu.async_copy(w_hbm.at[wid, u], w_v, gsem2)
            pltpu.make_async_copy(src_hbm.at[wid, u], src_v, gsem0).wait()
            pltpu.make_async_copy(dst_hbm.at[wid, u], dst_v, gsem1).wait()
            pltpu.make_async_copy(w_hbm.at[wid, u], w_v, gsem2).wait()

            pltpu.async_copy(support_hbm.at[src_v.at[0]], rows0, gsem0)

            bufs = (rows0, rows1, rows2)
            gsems = (gsem0, gsem1, gsem2)
            ssems = (ssem0, ssem1, ssem2)

            def triple_body(m, carry2):
                for k in range(3):
                    c = 3 * m + k
                    nk = (k + 1) % 3

                    if k == 2:
                        pltpu.make_async_copy(
                            bufs[nk], acc.at[dst_v.at[c - 2]],
                            ssems[nk]).wait()
                    else:
                        @pl.when(m >= 1)
                        def _recycle(c=c, nk=nk):
                            pltpu.make_async_copy(
                                bufs[nk], acc.at[dst_v.at[c - 2]],
                                ssems[nk]).wait()

                    pltpu.async_copy(support_hbm.at[src_v.at[c + 1]],
                                     bufs[nk], gsems[nk])

                    pltpu.make_async_copy(support_hbm.at[src_v.at[c]],
                                          bufs[k], gsems[k]).wait()
                    scale(bufs[k], c)
                    pltpu.async_copy(bufs[k], acc.at[dst_v.at[c]],
                                     ssems[k], add=True)
                return carry2

            lax.fori_loop(0, (_SCH - 1) // 3, triple_body, 0)

            t = _SCH - 1
            pltpu.make_async_copy(support_hbm.at[src_v.at[t]],
                                  rows0, gsem0).wait()
            scale(rows0, t)
            pltpu.async_copy(rows0, acc.at[dst_v.at[t]], ssem0, add=True)
            pltpu.make_async_copy(rows1, acc.at[dst_v.at[t - 2]],
                                  ssem1).wait()
            pltpu.make_async_copy(rows2, acc.at[dst_v.at[t - 1]],
                                  ssem2).wait()
            pltpu.make_async_copy(rows0, acc.at[dst_v.at[t]], ssem0).wait()
            return carry

        lax.fori_loop(0, _NSUP, super_body, 0)

        plsc.subcore_barrier()

        @pl.when(s < 10)
        def _drain():
            pltpu.sync_copy(acc.at[pl.ds(s * 1000, 1000)],
                            out_hbm.at[c, pl.ds(s * 1000, 1000)])

    return k(support, src, dst, wts)


def kernel(edge_index, edge_weight, input_feature, W):
    blk = N // 10
    support = pl.pallas_call(
        _mm_body,
        grid=(10,),
        in_specs=[pl.BlockSpec((blk, D), lambda i: (i, 0)),
                  pl.BlockSpec((D, D), lambda i: (0, 0))],
        out_specs=pl.BlockSpec((blk, D), lambda i: (i, 0)),
        out_shape=jax.ShapeDtypeStruct((N, D), jnp.float32),
    )(input_feature, W)

    src = edge_index[0].reshape(_NW, _NSUP, _SCH, _K)
    dst = edge_index[1].reshape(_NW, _NSUP, _SCH, _K)
    w = edge_weight.reshape(_NW, _NSUP, _SCH * _K)

    partials = _sc_scatter(support, src, dst, w)

    out = pl.pallas_call(
        _add_body,
        grid=(10,),
        in_specs=[pl.BlockSpec((blk, D), lambda i: (i, 0)),
                  pl.BlockSpec((blk, D), lambda i: (i, 0))],
        out_specs=pl.BlockSpec((blk, D), lambda i: (i, 0)),
        out_shape=jax.ShapeDtypeStruct((N, D), jnp.float32),
    )(partials[0], partials[1])
    return out


# speedup vs baseline: 2.6378x; 1.0906x over previous
import functools

import jax
import jax.numpy as jnp
from jax import lax
from jax.experimental import pallas as pl
from jax.experimental.pallas import tpu as pltpu
from jax.experimental.pallas import tpu_sc as plsc

N = 10000
E = 320000
D = 128

_NC = 2
_NS = 16
_NW = _NC * _NS
_EPW = E // _NW
_K = 80
_SCH = 25
_NSUP = _EPW // (_SCH * _K)
_L = 16


def _addmm_body(p_ref, w_ref, o_ref):
    o_ref[...] = jnp.dot(p_ref[0] + p_ref[1], w_ref[...],
                         preferred_element_type=jnp.float32)


def _sc_scatter(support, src, dst, wts):
    mesh = plsc.VectorSubcoreMesh(core_axis_name="c", subcore_axis_name="s")

    @functools.partial(
        pl.kernel,
        mesh=mesh,
        out_type=jax.ShapeDtypeStruct((_NC, N, D), jnp.float32),
        scratch_types=[
            pltpu.VMEM((_SCH, _K), jnp.int32),
            pltpu.VMEM((_SCH, _K), jnp.int32),
            pltpu.VMEM((_SCH * _K,), jnp.float32),
            pltpu.VMEM((_K, D), jnp.float32),
            pltpu.VMEM((_K, D), jnp.float32),
            pltpu.VMEM((_K, D), jnp.float32),
            pltpu.VMEM_SHARED((N, D), jnp.float32),
            pltpu.SemaphoreType.DMA,
            pltpu.SemaphoreType.DMA,
            pltpu.SemaphoreType.DMA,
            pltpu.SemaphoreType.DMA,
            pltpu.SemaphoreType.DMA,
            pltpu.SemaphoreType.DMA,
        ],
    )
    def k(support_hbm, src_hbm, dst_hbm, w_hbm, out_hbm,
          src_v, dst_v, w_v, rows0, rows1, rows2, acc,
          gsem0, gsem1, gsem2, ssem0, ssem1, ssem2):
        c = lax.axis_index("c")
        s = lax.axis_index("s")
        wid = c * _NS + s

        def zero_body(i, carry):
            for f in range(D // _L):
                rows0[i, pl.ds(f * _L, _L)] = jnp.zeros((_L,), jnp.float32)
            return carry

        lax.fori_loop(0, _K, zero_body, 0)

        @pl.when(s < 10)
        def _zero():
            for r in range(12):
                pltpu.sync_copy(rows0, acc.at[pl.ds(s * 1000 + r * _K, _K)])
            pltpu.sync_copy(rows0.at[pl.ds(0, 40)],
                            acc.at[pl.ds(s * 1000 + 960, 40)])

        plsc.subcore_barrier()

        def scale(rows, j):
            def group_body(g, inner):
                wvec = w_v[pl.ds(j * _K + g * _L, _L)]
                for lane in range(_L):
                    wb = jnp.full((_L,), wvec[lane])
                    row = g * _L + lane
                    for f in range(D // _L):
                        rows[row, pl.ds(f * _L, _L)] = (
                            rows[row, pl.ds(f * _L, _L)] * wb)
                return inner

            lax.fori_loop(0, _K // _L, group_body, 0)

        def super_body(u, carry):
            pltpu.async_copy(src_hbm.at[wid, u], src_v, gsem0)
            pltpu.async_copy(dst_hbm.at[wid, u], dst_v, gsem1)
            pltpu.async_copy(w_hbm.at[wid, u], w_v, gsem2)
            pltpu.make_async_copy(src_hbm.at[wid, u], src_v, gsem0).wait()
            pltpu.make_async_copy(dst_hbm.at[wid, u], dst_v, gsem1).wait()
            pltpu.make_async_copy(w_hbm.at[wid, u], w_v, gsem2).wait()

            pltpu.async_copy(support_hbm.at[src_v.at[0]], rows0, gsem0)

            bufs = (rows0, rows1, rows2)
            gsems = (gsem0, gsem1, gsem2)
            ssems = (ssem0, ssem1, ssem2)

            def triple_body(m, carry2):
                for k in range(3):
                    c = 3 * m + k
                    nk = (k + 1) % 3

                    if k == 2:
                        pltpu.make_async_copy(
                            bufs[nk], acc.at[dst_v.at[c - 2]],
                            ssems[nk]).wait()
                    else:
                        @pl.when(m >= 1)
                        def _recycle(c=c, nk=nk):
                            pltpu.make_async_copy(
                                bufs[nk], acc.at[dst_v.at[c - 2]],
                                ssems[nk]).wait()

                    pltpu.async_copy(support_hbm.at[src_v.at[c + 1]],
                                     bufs[nk], gsems[nk])

                    pltpu.make_async_copy(support_hbm.at[src_v.at[c]],
                                          bufs[k], gsems[k]).wait()
                    scale(bufs[k], c)
                    pltpu.async_copy(bufs[k], acc.at[dst_v.at[c]],
                                     ssems[k], add=True)
                return carry2

            lax.fori_loop(0, (_SCH - 1) // 3, triple_body, 0)

            t = _SCH - 1
            pltpu.make_async_copy(support_hbm.at[src_v.at[t]],
                                  rows0, gsem0).wait()
            scale(rows0, t)
            pltpu.async_copy(rows0, acc.at[dst_v.at[t]], ssem0, add=True)
            pltpu.make_async_copy(rows1, acc.at[dst_v.at[t - 2]],
                                  ssem1).wait()
            pltpu.make_async_copy(rows2, acc.at[dst_v.at[t - 1]],
                                  ssem2).wait()
            pltpu.make_async_copy(rows0, acc.at[dst_v.at[t]], ssem0).wait()
            return carry

        lax.fori_loop(0, _NSUP, super_body, 0)

        plsc.subcore_barrier()

        @pl.when(s < 10)
        def _drain():
            pltpu.sync_copy(acc.at[pl.ds(s * 1000, 1000)],
                            out_hbm.at[c, pl.ds(s * 1000, 1000)])

    return k(support, src, dst, wts)


def kernel(edge_index, edge_weight, input_feature, W):
    src = edge_index[0].reshape(_NW, _NSUP, _SCH, _K)
    dst = edge_index[1].reshape(_NW, _NSUP, _SCH, _K)
    w = edge_weight.reshape(_NW, _NSUP, _SCH * _K)

    partials = _sc_scatter(input_feature, src, dst, w)

    blk = N // 10
    out = pl.pallas_call(
        _addmm_body,
        grid=(10,),
        in_specs=[pl.BlockSpec((2, blk, D), lambda i: (0, i, 0)),
                  pl.BlockSpec((D, D), lambda i: (0, 0))],
        out_specs=pl.BlockSpec((blk, D), lambda i: (i, 0)),
        out_shape=jax.ShapeDtypeStruct((N, D), jnp.float32),
    )(partials, W)
    return out
